# Initial kernel scaffold; baseline (speedup 1.0000x reference)
import jax
import jax.numpy as jnp
from jax.experimental import pallas as pl


def kernel(ids, W):
    raise NotImplementedError("write your pallas kernel here")


# speedup vs baseline: 3.7658x; 3.7658x over previous
import functools

import jax
import jax.numpy as jnp
from jax import lax
from jax.experimental import pallas as pl
from jax.experimental.pallas import tpu as pltpu
from jax.experimental.pallas import tpu_sc as plsc

EMBED_D = 64
NUM_CORES = 2
NUM_SUBCORES = 16
NUM_WORKERS = NUM_CORES * NUM_SUBCORES
CHUNK = 128


def _make_lookup(total_rows: int):
  rows_per_w = total_rows // NUM_WORKERS
  n_chunks = rows_per_w // CHUNK
  assert rows_per_w % CHUNK == 0

  mesh = plsc.VectorSubcoreMesh(
      core_axis_name="c", subcore_axis_name="s", num_cores=NUM_CORES)

  @functools.partial(
      pl.kernel,
      out_type=jax.ShapeDtypeStruct((total_rows, EMBED_D), jnp.float32),
      mesh=mesh,
      compiler_params=pltpu.CompilerParams(use_tc_tiling_on_sc=False),
      scratch_types=[
          pltpu.VMEM((CHUNK,), jnp.int32),
          pltpu.VMEM((CHUNK, EMBED_D), jnp.float32),
          pltpu.SemaphoreType.DMA,
      ],
  )
  def lookup(table_hbm, idx_hbm, out_hbm, idx_v, rows_v, sem):
    wid = lax.axis_index("s") * NUM_CORES + lax.axis_index("c")
    base = wid * rows_per_w

    def body(i, carry):
      off = base + i * CHUNK
      pltpu.sync_copy(idx_hbm.at[pl.ds(off, CHUNK)], idx_v)
      pltpu.async_copy(table_hbm.at[idx_v], rows_v, sem).wait()
      pltpu.sync_copy(rows_v, out_hbm.at[pl.ds(off, CHUNK)])
      return carry

    lax.fori_loop(0, n_chunks, body, 0)

  return lookup


def kernel(ids, W):
  flat_ids = ids.reshape(-1).astype(jnp.int32)
  out = _make_lookup(flat_ids.shape[0])(W, flat_ids)
  return out.reshape(ids.shape + (EMBED_D,))


# speedup vs baseline: 4.6205x; 1.2270x over previous
import functools

import jax
import jax.numpy as jnp
from jax import lax
from jax.experimental import pallas as pl
from jax.experimental.pallas import tpu as pltpu
from jax.experimental.pallas import tpu_sc as plsc

EMBED_D = 64
NUM_CORES = 2
NUM_SUBCORES = 16
NUM_WORKERS = NUM_CORES * NUM_SUBCORES
CHUNK = 128
K = 5
GROUP = CHUNK * K


def _make_lookup(total_rows: int):
  rows_per_w = total_rows // NUM_WORKERS
  idx_rows_per_w = rows_per_w // CHUNK
  n_groups = idx_rows_per_w // K
  assert rows_per_w % (CHUNK * K) == 0 and n_groups % 2 == 0

  mesh = plsc.VectorSubcoreMesh(
      core_axis_name="c", subcore_axis_name="s", num_cores=NUM_CORES)

  @functools.partial(
      pl.kernel,
      out_type=jax.ShapeDtypeStruct((total_rows, EMBED_D), jnp.float32),
      mesh=mesh,
      compiler_params=pltpu.CompilerParams(use_tc_tiling_on_sc=False),
      scratch_types=[
          pltpu.VMEM((idx_rows_per_w, CHUNK), jnp.int32),
          pltpu.VMEM((GROUP, EMBED_D), jnp.float32),
          pltpu.VMEM((GROUP, EMBED_D), jnp.float32),
          pltpu.SemaphoreType.DMA,
          pltpu.SemaphoreType.DMA,
          pltpu.SemaphoreType.DMA,
      ],
  )
  def lookup(table_hbm, idx_hbm, out_hbm, idx_v, rows0, rows1, gsem, ssem0,
             ssem1):
    wid = lax.axis_index("s") * NUM_CORES + lax.axis_index("c")
    idx_base = wid * idx_rows_per_w
    out_base = wid * rows_per_w

    pltpu.sync_copy(idx_hbm.at[pl.ds(idx_base, idx_rows_per_w)], idx_v)

    def body(h, carry):
      for b, (rows_v, ssem) in enumerate(((rows0, ssem0), (rows1, ssem1))):
        g = h * 2 + b
        out_off = out_base + g * GROUP

        @pl.when(h > 0)
        def _wait_prev_store():
          prev_off = out_base + (g - 2) * GROUP
          pltpu.make_async_copy(
              rows_v, out_hbm.at[pl.ds(prev_off, GROUP)], ssem).wait()

        for j in range(K):
          pltpu.async_copy(
              table_hbm.at[idx_v.at[g * K + j]],
              rows_v.at[pl.ds(j * CHUNK, CHUNK)], gsem)
        for j in range(K):
          pltpu.make_async_copy(
              table_hbm.at[idx_v.at[g * K + j]],
              rows_v.at[pl.ds(j * CHUNK, CHUNK)], gsem).wait()

        pltpu.async_copy(rows_v, out_hbm.at[pl.ds(out_off, GROUP)], ssem)
      return carry

    lax.fori_loop(0, n_groups // 2, body, 0)

    pltpu.make_async_copy(
        rows0, out_hbm.at[pl.ds(out_base + (n_groups - 2) * GROUP, GROUP)],
        ssem0).wait()
    pltpu.make_async_copy(
        rows1, out_hbm.at[pl.ds(out_base + (n_groups - 1) * GROUP, GROUP)],
        ssem1).wait()

  return lookup


def kernel(ids, W):
  flat_ids = ids.reshape(-1).astype(jnp.int32)
  total_rows = flat_ids.shape[0]
  idx2d = flat_ids.reshape(total_rows // CHUNK, CHUNK)
  out = _make_lookup(total_rows)(W, idx2d)
  return out.reshape(ids.shape + (EMBED_D,))
